# baseline (speedup 1.0000x reference)
import functools

import jax
import jax.numpy as jnp
from jax import lax
from jax.experimental import pallas as pl
from jax.experimental.pallas import tpu as pltpu
from jax.experimental.pallas import tpu_sc as plsc

_VOCAB = 1000000
_DIM = 64
_PROB = 0.1


@functools.cache
def _make_sc_gather(V, D, B):
    info = plsc.get_sparse_core_info()
    NC, NS, L = info.num_cores, info.num_subcores, info.num_lanes
    NW = NC * NS
    assert D % L == 0 and B % (8 * NW) == 0
    b_per_w = B // NW
    C = 512
    assert b_per_w % C == 0
    n_chunks = b_per_w // C
    mesh = plsc.VectorSubcoreMesh(core_axis_name="c", subcore_axis_name="s")

    @functools.partial(
        pl.kernel,
        mesh=mesh,
        out_type=jax.ShapeDtypeStruct((B, D), jnp.float32),
        scratch_types=[
            pltpu.VMEM((C,), jnp.int32),
            pltpu.VMEM((C,), jnp.float32),
            pltpu.VMEM((C, D), jnp.float32),
            pltpu.SemaphoreType.DMA,
            pltpu.SemaphoreType.DMA,
        ],
        compiler_params=pltpu.CompilerParams(use_tc_tiling_on_sc=False),
    )
    def k(w_hbm, keep_hbm, idx_hbm, out_hbm, idx_v, keep_v, rows_v, sem_r, sem_k):
        wid = lax.axis_index("s") * NC + lax.axis_index("c")
        base = wid * b_per_w

        def chunk_body(i, carry):
            off = base + i * C
            pltpu.sync_copy(idx_hbm.at[pl.ds(off, C)], idx_v)
            cp_r = pltpu.async_copy(w_hbm.at[idx_v], rows_v, sem_r)
            cp_k = pltpu.async_copy(keep_hbm.at[idx_v], keep_v, sem_k)
            cp_k.wait()
            cp_r.wait()

            def grp_body(g, c2):
                keep16 = keep_v[pl.ds(g * L, L)]
                for e in range(L):
                    r = g * L + e
                    k16 = jnp.broadcast_to(keep16[e], (L,))
                    for j in range(D // L):
                        rows_v[r, pl.ds(j * L, L)] = rows_v[r, pl.ds(j * L, L)] * k16
                return c2

            lax.fori_loop(0, C // L, grp_body, 0)
            pltpu.sync_copy(rows_v, out_hbm.at[pl.ds(off, C)])
            return carry

        lax.fori_loop(0, n_chunks, chunk_body, 0)

    return k


def kernel(x, W):
    B, H = x.shape
    V, D = W.shape
    keep = jax.random.bernoulli(
        jax.random.key(42), 1.0 - _PROB, (V, 1)
    ).astype(W.dtype) / (1.0 - _PROB)
    keep = keep.reshape(V)
    idx = x.reshape(B * H).astype(jnp.int32)
    out = _make_sc_gather(V, D, B * H)(W, keep, idx)
    return out.reshape(B, H, D)


# speedup vs baseline: 1.1047x; 1.1047x over previous
import functools

import jax
import jax.numpy as jnp
from jax import lax
from jax.experimental import pallas as pl
from jax.experimental.pallas import tpu as pltpu
from jax.experimental.pallas import tpu_sc as plsc

_VOCAB = 1000000
_DIM = 64
_PROB = 0.1


@functools.cache
def _make_sc_gather(V, D, B):
    info = plsc.get_sparse_core_info()
    NC, NS, L = info.num_cores, info.num_subcores, info.num_lanes
    NW = NC * NS
    assert D % L == 0 and B % (8 * NW) == 0
    b_per_w = B // NW
    C = 512
    assert b_per_w % (2 * C) == 0
    n_chunks = b_per_w // C
    mesh = plsc.VectorSubcoreMesh(core_axis_name="c", subcore_axis_name="s")

    @functools.partial(
        pl.kernel,
        mesh=mesh,
        out_type=jax.ShapeDtypeStruct((B, D), jnp.float32),
        scratch_types=[
            pltpu.VMEM((b_per_w,), jnp.int32),
            pltpu.VMEM((C,), jnp.float32),
            pltpu.VMEM((C,), jnp.float32),
            pltpu.VMEM((C, D), jnp.float32),
            pltpu.VMEM((C, D), jnp.float32),
            pltpu.SemaphoreType.DMA,
            pltpu.SemaphoreType.DMA,
        ],
        compiler_params=pltpu.CompilerParams(use_tc_tiling_on_sc=False),
    )
    def k(w_hbm, keep_hbm, idx_hbm, out_hbm,
          idx_all, keep0, keep1, rows0, rows1, sem0, sem1):
        wid = lax.axis_index("s") * NC + lax.axis_index("c")
        base = wid * b_per_w
        pltpu.sync_copy(idx_hbm.at[pl.ds(base, b_per_w)], idx_all)
        rows = (rows0, rows1)
        keeps = (keep0, keep1)
        sems = (sem0, sem1)

        def fire(i, b):
            sl = idx_all.at[pl.ds(i * C, C)]
            pltpu.async_copy(w_hbm.at[sl], rows[b], sems[b])
            pltpu.async_copy(keep_hbm.at[sl], keeps[b], sems[b])

        def scale(b):
            rv, kv = rows[b], keeps[b]

            def grp_body(g, c2):
                keep16 = kv[pl.ds(g * L, L)]
                for e in range(L):
                    r = g * L + e
                    k16 = jnp.broadcast_to(keep16[e], (L,))
                    for j in range(D // L):
                        rv[r, pl.ds(j * L, L)] = rv[r, pl.ds(j * L, L)] * k16
                return c2

            lax.fori_loop(0, C // L, grp_body, 0)

        def half(i, b):
            @pl.when(i + 1 < n_chunks)
            def _():
                fire(i + 1, 1 - b)

            pltpu.make_async_copy(
                w_hbm.at[pl.ds(0, C)], rows[b], sems[b]).wait()
            pltpu.make_async_copy(
                keep_hbm.at[pl.ds(0, C)], keeps[b], sems[b]).wait()
            scale(b)
            pltpu.sync_copy(rows[b], out_hbm.at[pl.ds(base + i * C, C)])

        fire(0, 0)

        def pair_body(t, carry):
            half(2 * t, 0)
            half(2 * t + 1, 1)
            return carry

        lax.fori_loop(0, n_chunks // 2, pair_body, 0)

    return k


def kernel(x, W):
    B, H = x.shape
    V, D = W.shape
    keep = jax.random.bernoulli(
        jax.random.key(42), 1.0 - _PROB, (V, 1)
    ).astype(W.dtype) / (1.0 - _PROB)
    keep = keep.reshape(V)
    idx = x.reshape(B * H).astype(jnp.int32)
    out = _make_sc_gather(V, D, B * H)(W, keep, idx)
    return out.reshape(B, H, D)
